# Initial kernel scaffold; baseline (speedup 1.0000x reference)
import jax
import jax.numpy as jnp
from jax.experimental import pallas as pl


def kernel(x, edge_index, batch, W1, b1, W2, b2, W3, b3):
    raise NotImplementedError("write your pallas kernel here")


# speedup vs baseline: 15.1491x; 15.1491x over previous
import functools

import jax
import jax.numpy as jnp
from jax import lax
from jax.experimental import pallas as pl
from jax.experimental.pallas import tpu as pltpu
from jax.experimental.pallas import tpu_sc as plsc

N_NODES = 10000
E_EDGES = 320000
D_IN = 128
H_DIM = 64
C_OUT = 6
G_GRAPHS = 64

NUM_CORES = 2
NUM_SUBCORES = 16
NUM_WORKERS = NUM_CORES * NUM_SUBCORES

NP = 10240
STRIPE = NP // NUM_SUBCORES
CHUNK = 128
EPT = 10112
NCHUNK = EPT // CHUNK
EPAD = EPT * NUM_WORKERS

_mesh = plsc.VectorSubcoreMesh(core_axis_name="c", subcore_axis_name="s")


@functools.partial(
    pl.kernel,
    mesh=_mesh,
    out_type=jax.ShapeDtypeStruct((NUM_CORES, NP), jnp.float32),
    scratch_types=[
        pltpu.VMEM((CHUNK,), jnp.int32),
        pltpu.VMEM((CHUNK,), jnp.float32),
        pltpu.VMEM((STRIPE,), jnp.float32),
        pltpu.VMEM_SHARED((NP,), jnp.float32),
        pltpu.SemaphoreType.DMA,
    ],
)
def _sc_degree(dst_hbm, out_hbm, dst_v, ones_v, zero_v, acc_sh, sem):
    c = lax.axis_index("c")
    s = lax.axis_index("s")
    wid = s * NUM_CORES + c

    one16 = jnp.full((16,), 1.0, jnp.float32)
    zero16 = jnp.zeros((16,), jnp.float32)
    for j in range(CHUNK // 16):
        ones_v[pl.ds(j * 16, 16)] = one16

    def zfill(i, carry):
        zero_v[pl.ds(i * 16, 16)] = zero16
        return carry

    lax.fori_loop(0, STRIPE // 16, zfill, 0)
    pltpu.sync_copy(zero_v, acc_sh.at[pl.ds(s * STRIPE, STRIPE)])
    plsc.subcore_barrier()

    def body(i, carry):
        base = wid * EPT + i * CHUNK
        pltpu.sync_copy(dst_hbm.at[pl.ds(base, CHUNK)], dst_v)
        pltpu.sync_copy(ones_v, acc_sh.at[dst_v], add=True)
        return carry

    lax.fori_loop(0, NCHUNK, body, 0)
    plsc.subcore_barrier()
    pltpu.sync_copy(acc_sh.at[pl.ds(s * STRIPE, STRIPE)],
                    out_hbm.at[c, pl.ds(s * STRIPE, STRIPE)])


@functools.partial(
    pl.kernel,
    mesh=_mesh,
    out_type=jax.ShapeDtypeStruct((NUM_CORES, NP, H_DIM), jnp.float32),
    scratch_types=[
        pltpu.VMEM((CHUNK,), jnp.int32),
        pltpu.VMEM((CHUNK,), jnp.int32),
        pltpu.VMEM((CHUNK, H_DIM), jnp.float32),
        pltpu.VMEM((STRIPE, H_DIM), jnp.float32),
        pltpu.VMEM_SHARED((NP, H_DIM), jnp.float32),
        pltpu.SemaphoreType.DMA,
    ],
    compiler_params=pltpu.CompilerParams(use_tc_tiling_on_sc=False),
)
def _sc_edge_pass(u_hbm, src_hbm, dst_hbm, out_hbm,
                  src_v, dst_v, rows_v, zero_v, acc_sh, sem):
    c = lax.axis_index("c")
    s = lax.axis_index("s")
    wid = s * NUM_CORES + c

    zero16 = jnp.zeros((16,), jnp.float32)

    def zfill(i, carry):
        for j in range(H_DIM // 16):
            zero_v[i, pl.ds(j * 16, 16)] = zero16
        return carry

    lax.fori_loop(0, STRIPE, zfill, 0)
    pltpu.sync_copy(zero_v, acc_sh.at[pl.ds(s * STRIPE, STRIPE)])
    plsc.subcore_barrier()

    def body(i, carry):
        base = wid * EPT + i * CHUNK
        pltpu.sync_copy(src_hbm.at[pl.ds(base, CHUNK)], src_v)
        pltpu.sync_copy(dst_hbm.at[pl.ds(base, CHUNK)], dst_v)
        pltpu.async_copy(u_hbm.at[src_v], rows_v, sem).wait()
        pltpu.sync_copy(rows_v, acc_sh.at[dst_v], add=True)
        return carry

    lax.fori_loop(0, NCHUNK, body, 0)
    plsc.subcore_barrier()
    pltpu.sync_copy(acc_sh.at[pl.ds(s * STRIPE, STRIPE)],
                    out_hbm.at[c, pl.ds(s * STRIPE, STRIPE)])


def _tc_first_body(x_ref, w1_ref, deg_ref, u_ref, dinv_ref):
    deg = deg_ref[0][:, None] + deg_ref[1][:, None] + 1.0
    dinv = lax.rsqrt(deg)
    z = jnp.dot(x_ref[...], w1_ref[...], preferred_element_type=jnp.float32)
    u_ref[...] = z * dinv
    dinv_ref[...] = dinv


def _tc_mid_body(p_ref, u1_ref, dinv_ref, b1_ref, w2_ref, u2_ref):
    ssum = p_ref[0] + p_ref[1] + u1_ref[...]
    h = jnp.maximum(ssum * dinv_ref[...] + b1_ref[...], 0.0)
    u2_ref[...] = jnp.dot(h * dinv_ref[...], w2_ref[...],
                          preferred_element_type=jnp.float32)


def _tc_tail_body(p_ref, u2_ref, dinv_ref, b2_ref, batch_ref, w3_ref, b3_ref,
                  out_ref):
    ssum = p_ref[0] + p_ref[1] + u2_ref[...]
    h = jnp.maximum(ssum * dinv_ref[...] + b2_ref[...], 0.0)
    gid = lax.broadcasted_iota(jnp.int32, (NP, G_GRAPHS), 1)
    onehot = (batch_ref[...] == gid).astype(jnp.float32)
    gsum = lax.dot_general(onehot, h, (((0,), (0,)), ((), ())),
                           preferred_element_type=jnp.float32)
    cnt = jnp.sum(onehot, axis=0)[:, None]
    g = gsum / jnp.maximum(cnt, 1.0)
    out_ref[...] = jnp.dot(g, w3_ref[...],
                           preferred_element_type=jnp.float32) + b3_ref[...]


_tc_first = pl.pallas_call(
    _tc_first_body,
    out_shape=(jax.ShapeDtypeStruct((NP, H_DIM), jnp.float32),
               jax.ShapeDtypeStruct((NP, 1), jnp.float32)),
)

_tc_mid = pl.pallas_call(
    _tc_mid_body,
    out_shape=jax.ShapeDtypeStruct((NP, H_DIM), jnp.float32),
)

_tc_tail = pl.pallas_call(
    _tc_tail_body,
    out_shape=jax.ShapeDtypeStruct((G_GRAPHS, C_OUT), jnp.float32),
)


def kernel(x, edge_index, batch, W1, b1, W2, b2, W3, b3):
    pad_e = EPAD - E_EDGES
    srcp = jnp.concatenate(
        [edge_index[0], jnp.zeros((pad_e,), jnp.int32)])
    dstp = jnp.concatenate(
        [edge_index[1], jnp.full((pad_e,), N_NODES, jnp.int32)])
    x_p = jnp.concatenate(
        [x, jnp.zeros((NP - N_NODES, D_IN), jnp.float32)])
    batch_p = jnp.concatenate(
        [batch, jnp.full((NP - N_NODES,), G_GRAPHS, jnp.int32)])[:, None]

    deg_parts = _sc_degree(dstp)
    u1, dinv = _tc_first(x_p, W1, deg_parts)
    p1 = _sc_edge_pass(u1, srcp, dstp)
    u2 = _tc_mid(p1, u1, dinv, b1[None, :], W2)
    p2 = _sc_edge_pass(u2, srcp, dstp)
    return _tc_tail(p2, u2, dinv, b2[None, :], batch_p, W3, b3[None, :])


# speedup vs baseline: 16.6915x; 1.1018x over previous
import functools

import jax
import jax.numpy as jnp
from jax import lax
from jax.experimental import pallas as pl
from jax.experimental.pallas import tpu as pltpu
from jax.experimental.pallas import tpu_sc as plsc

N_NODES = 10000
E_EDGES = 320000
D_IN = 128
H_DIM = 64
C_OUT = 6
G_GRAPHS = 64

NUM_CORES = 2
NUM_SUBCORES = 16
NUM_WORKERS = NUM_CORES * NUM_SUBCORES

NP = 10240
STRIPE = NP // NUM_SUBCORES
CHUNK = 128
SUB = 4
BLK = SUB * CHUNK
NBLK = 20
EPT = NBLK * BLK
EPAD = EPT * NUM_WORKERS
NCHUNK = EPT // CHUNK

_mesh = plsc.VectorSubcoreMesh(core_axis_name="c", subcore_axis_name="s")


@functools.partial(
    pl.kernel,
    mesh=_mesh,
    out_type=jax.ShapeDtypeStruct((NUM_CORES, NP), jnp.float32),
    scratch_types=[
        pltpu.VMEM((CHUNK,), jnp.int32),
        pltpu.VMEM((CHUNK,), jnp.float32),
        pltpu.VMEM((STRIPE,), jnp.float32),
        pltpu.VMEM_SHARED((NP,), jnp.float32),
        pltpu.SemaphoreType.DMA,
    ],
)
def _sc_degree(dst_hbm, out_hbm, dst_v, ones_v, zero_v, acc_sh, sem):
    c = lax.axis_index("c")
    s = lax.axis_index("s")
    wid = s * NUM_CORES + c

    one16 = jnp.full((16,), 1.0, jnp.float32)
    zero16 = jnp.zeros((16,), jnp.float32)
    for j in range(CHUNK // 16):
        ones_v[pl.ds(j * 16, 16)] = one16

    def zfill(i, carry):
        zero_v[pl.ds(i * 16, 16)] = zero16
        return carry

    lax.fori_loop(0, STRIPE // 16, zfill, 0)
    pltpu.sync_copy(zero_v, acc_sh.at[pl.ds(s * STRIPE, STRIPE)])
    plsc.subcore_barrier()

    def body(i, carry):
        base = wid * EPT + i * CHUNK
        pltpu.sync_copy(dst_hbm.at[pl.ds(base, CHUNK)], dst_v)
        pltpu.sync_copy(ones_v, acc_sh.at[dst_v], add=True)
        return carry

    lax.fori_loop(0, NCHUNK, body, 0)
    plsc.subcore_barrier()
    pltpu.sync_copy(acc_sh.at[pl.ds(s * STRIPE, STRIPE)],
                    out_hbm.at[c, pl.ds(s * STRIPE, STRIPE)])


@functools.partial(
    pl.kernel,
    mesh=_mesh,
    out_type=jax.ShapeDtypeStruct((NUM_CORES, NP, H_DIM), jnp.float32),
    scratch_types=[
        pltpu.VMEM((2, SUB, CHUNK), jnp.int32),
        pltpu.VMEM((2, SUB, CHUNK), jnp.int32),
        pltpu.VMEM((SUB, CHUNK, H_DIM), jnp.float32),
        pltpu.VMEM((SUB, CHUNK, H_DIM), jnp.float32),
        pltpu.VMEM((CHUNK, H_DIM), jnp.float32),
        pltpu.VMEM_SHARED((NP, H_DIM), jnp.float32),
        pltpu.SemaphoreType.DMA,
        pltpu.SemaphoreType.DMA,
        pltpu.SemaphoreType.DMA,
        pltpu.SemaphoreType.DMA,
    ],
    compiler_params=pltpu.CompilerParams(use_tc_tiling_on_sc=False),
)
def _sc_edge_pass(u_hbm, eidx_hbm, out_hbm, ibuf0, ibuf1, rows0, rows1,
                  zrow, acc_sh, isem0, isem1, gsem0, gsem1):
    c = lax.axis_index("c")
    s = lax.axis_index("s")
    wid = s * NUM_CORES + c
    blk0 = wid * NBLK

    ibuf = (ibuf0, ibuf1)
    rows = (rows0, rows1)
    isem = (isem0, isem1)
    gsem = (gsem0, gsem1)

    zero16 = jnp.zeros((16,), jnp.float32)

    def zfill(i, carry):
        for j in range(H_DIM // 16):
            zrow[i, pl.ds(j * 16, 16)] = zero16
        return carry

    lax.fori_loop(0, CHUNK, zfill, 0)
    for t in range(STRIPE // CHUNK):
        pltpu.sync_copy(zrow, acc_sh.at[pl.ds(s * STRIPE + t * CHUNK, CHUNK)])
    plsc.subcore_barrier()

    def start_idx(b):
        return pltpu.async_copy(eidx_hbm.at[blk0 + b], ibuf[b % 2],
                                isem[b % 2])

    def fire_gathers(b):
        return [pltpu.async_copy(u_hbm.at[ibuf[b % 2].at[0, j]],
                                 rows[b % 2].at[j], gsem[b % 2])
                for j in range(SUB)]

    h_idx = [start_idx(0), start_idx(1)]
    h_idx[0].wait()
    pending = {0: fire_gathers(0)}
    for b in range(NBLK):
        if b + 1 < NBLK:
            h_idx[(b + 1) % 2].wait()
            pending[b + 1] = fire_gathers(b + 1)
        for h in pending.pop(b):
            h.wait()
        for j in range(SUB):
            pltpu.sync_copy(rows[b % 2].at[j],
                            acc_sh.at[ibuf[b % 2].at[1, j]], add=True)
        if b + 2 < NBLK:
            h_idx[b % 2] = start_idx(b + 2)
    plsc.subcore_barrier()
    pltpu.sync_copy(acc_sh.at[pl.ds(s * STRIPE, STRIPE)],
                    out_hbm.at[c, pl.ds(s * STRIPE, STRIPE)])


def _tc_first_body(x_ref, w1_ref, deg_ref, u_ref, dinv_ref):
    deg = deg_ref[0][:, None] + deg_ref[1][:, None] + 1.0
    dinv = lax.rsqrt(deg)
    z = jnp.dot(x_ref[...], w1_ref[...], preferred_element_type=jnp.float32)
    u_ref[...] = z * dinv
    dinv_ref[...] = dinv


def _tc_mid_body(p_ref, u1_ref, dinv_ref, b1_ref, w2_ref, u2_ref):
    ssum = p_ref[0] + p_ref[1] + u1_ref[...]
    h = jnp.maximum(ssum * dinv_ref[...] + b1_ref[...], 0.0)
    u2_ref[...] = jnp.dot(h * dinv_ref[...], w2_ref[...],
                          preferred_element_type=jnp.float32)


def _tc_tail_body(p_ref, u2_ref, dinv_ref, b2_ref, batch_ref, w3_ref, b3_ref,
                  out_ref):
    ssum = p_ref[0] + p_ref[1] + u2_ref[...]
    h = jnp.maximum(ssum * dinv_ref[...] + b2_ref[...], 0.0)
    gid = lax.broadcasted_iota(jnp.int32, (NP, G_GRAPHS), 1)
    onehot = (batch_ref[...] == gid).astype(jnp.float32)
    gsum = lax.dot_general(onehot, h, (((0,), (0,)), ((), ())),
                           preferred_element_type=jnp.float32)
    cnt = jnp.sum(onehot, axis=0)[:, None]
    g = gsum / jnp.maximum(cnt, 1.0)
    out_ref[...] = jnp.dot(g, w3_ref[...],
                           preferred_element_type=jnp.float32) + b3_ref[...]


_tc_first = pl.pallas_call(
    _tc_first_body,
    out_shape=(jax.ShapeDtypeStruct((NP, H_DIM), jnp.float32),
               jax.ShapeDtypeStruct((NP, 1), jnp.float32)),
)

_tc_mid = pl.pallas_call(
    _tc_mid_body,
    out_shape=jax.ShapeDtypeStruct((NP, H_DIM), jnp.float32),
)

_tc_tail = pl.pallas_call(
    _tc_tail_body,
    out_shape=jax.ShapeDtypeStruct((G_GRAPHS, C_OUT), jnp.float32),
)


def kernel(x, edge_index, batch, W1, b1, W2, b2, W3, b3):
    pad_e = EPAD - E_EDGES
    srcp = jnp.concatenate(
        [edge_index[0], jnp.zeros((pad_e,), jnp.int32)])
    dstp = jnp.concatenate(
        [edge_index[1], jnp.full((pad_e,), N_NODES, jnp.int32)])
    x_p = jnp.concatenate(
        [x, jnp.zeros((NP - N_NODES, D_IN), jnp.float32)])
    batch_p = jnp.concatenate(
        [batch, jnp.full((NP - N_NODES,), G_GRAPHS, jnp.int32)])[:, None]

    eidx = jnp.stack([srcp.reshape(-1, SUB, CHUNK),
                      dstp.reshape(-1, SUB, CHUNK)], axis=1)

    deg_parts = _sc_degree(dstp)
    u1, dinv = _tc_first(x_p, W1, deg_parts)
    p1 = _sc_edge_pass(u1, eidx)
    u2 = _tc_mid(p1, u1, dinv, b1[None, :], W2)
    p2 = _sc_edge_pass(u2, eidx)
    return _tc_tail(p2, u2, dinv, b2[None, :], batch_p, W3, b3[None, :])
